# Initial kernel scaffold; baseline (speedup 1.0000x reference)
import jax
import jax.numpy as jnp
from jax.experimental import pallas as pl


def kernel(x_g, vertices_g, hyperedges_g, edge_features_g, x_h, vertices_h, hyperedges_h, edge_features_h, params):
    raise NotImplementedError("write your pallas kernel here")


# speedup vs baseline: 3.9060x; 3.9060x over previous
import functools

import jax
import jax.numpy as jnp
from jax import lax
from jax.experimental import pallas as pl
from jax.experimental.pallas import tpu as pltpu
from jax.experimental.pallas import tpu_sc as plsc

N_NODES = 10000
N_EDGES = 160000
D = 128
NC = 2
NS = 16
NW = NC * NS
B = 400


def _sc_mesh():
    return plsc.VectorSubcoreMesh(core_axis_name="c", subcore_axis_name="s")


def _sc_gather(table, idx):
    n = idx.shape[0]
    per = n // NW
    nb = per // B

    @functools.partial(
        pl.kernel, mesh=_sc_mesh(),
        out_type=jax.ShapeDtypeStruct((n, D), jnp.float32),
        scratch_types=[
            pltpu.VMEM((B,), jnp.int32),
            pltpu.VMEM((B, D), jnp.float32),
            pltpu.SemaphoreType.DMA,
        ],
    )
    def k(table_hbm, idx_hbm, out_hbm, idx_v, rows_v, sem):
        wid = lax.axis_index("s") * NC + lax.axis_index("c")
        base0 = wid * per
        for j in range(nb):
            base = base0 + j * B
            pltpu.sync_copy(idx_hbm.at[pl.ds(base, B)], idx_v)
            pltpu.async_copy(table_hbm.at[idx_v], rows_v, sem).wait()
            pltpu.sync_copy(rows_v, out_hbm.at[pl.ds(base, B)])

    return k(table, idx)


def _zero_zbuf(zbuf, rows):
    def zrow(r, _):
        for c8 in range(D // 16):
            zbuf[r, pl.ds(c8 * 16, 16)] = jnp.zeros((16,), jnp.float32)
        return _
    lax.fori_loop(0, rows, zrow, 0)


def _zero_slice(zbuf, zr, dst, r0, rows):
    for off in range(0, rows, zr):
        cnt = min(zr, rows - off)
        src = zbuf if cnt == zr else zbuf.at[pl.ds(0, cnt)]
        pltpu.sync_copy(src, dst.at[pl.ds(r0 + off, cnt)])


def _sc_gather_scatter(table, idx_src, idx_dst):
    n = idx_src.shape[0]
    bk = 200
    per = n // NW
    nb = per // bk
    acc_rows = 10240
    rpt = acc_rows // NS

    @functools.partial(
        pl.kernel, mesh=_sc_mesh(),
        out_type=jax.ShapeDtypeStruct((NC, acc_rows, D), jnp.float32),
        scratch_types=[
            pltpu.VMEM((bk,), jnp.int32),
            pltpu.VMEM((bk,), jnp.int32),
            pltpu.VMEM((bk, D), jnp.float32),
            pltpu.VMEM_SHARED((acc_rows, D), jnp.float32),
            pltpu.SemaphoreType.DMA,
        ],
    )
    def k(table_hbm, src_hbm, dst_hbm, out_hbm, idxs_v, idxd_v, rows_v,
          acc_sh, sem):
        c = lax.axis_index("c")
        s = lax.axis_index("s")
        wid = s * NC + c
        _zero_zbuf(rows_v, bk)
        r0 = s * rpt
        _zero_slice(rows_v, bk, acc_sh, r0, rpt)
        plsc.subcore_barrier()
        base0 = wid * per
        for j in range(nb):
            base = base0 + j * bk
            pltpu.sync_copy(src_hbm.at[pl.ds(base, bk)], idxs_v)
            pltpu.sync_copy(dst_hbm.at[pl.ds(base, bk)], idxd_v)
            pltpu.async_copy(table_hbm.at[idxs_v], rows_v, sem).wait()
            pltpu.sync_copy(rows_v, acc_sh.at[idxd_v], add=True)
        plsc.subcore_barrier()
        pltpu.sync_copy(acc_sh.at[pl.ds(r0, rpt)], out_hbm.at[c, pl.ds(r0, rpt)])

    return k(table, idx_src, idx_dst)


W_SEG = 6400
N_PASS = N_EDGES // W_SEG
SEG_PAD = NW * B
BND_LEN = 48


def _sc_segsum(table, v_pad, he_pad, bounds):
    zr = 80
    rpt = W_SEG // NS

    @functools.partial(
        pl.kernel, mesh=_sc_mesh(),
        out_type=jax.ShapeDtypeStruct((NC, N_EDGES, D), jnp.float32),
        scratch_types=[
            pltpu.VMEM((BND_LEN,), jnp.int32),
            pltpu.VMEM((B,), jnp.int32),
            pltpu.VMEM((B,), jnp.int32),
            pltpu.VMEM((B,), jnp.int32),
            pltpu.VMEM((B, D), jnp.float32),
            pltpu.VMEM((zr, D), jnp.float32),
            pltpu.VMEM_SHARED((W_SEG + 8, D), jnp.float32),
            pltpu.SemaphoreType.DMA,
        ],
    )
    def k(table_hbm, v_hbm, he_hbm, bounds_hbm, out_hbm, bnd_v, idxv_v,
          idxe_v, idxl_v, rows_v, zbuf, acc_sh, sem):
        c = lax.axis_index("c")
        s = lax.axis_index("s")
        wid = s * NC + c
        _zero_zbuf(zbuf, zr)
        pltpu.sync_copy(bounds_hbm, bnd_v)

        def _scalar(p):
            return bnd_v[pl.ds(p, 16)][0]

        r0 = s * rpt
        for p in range(N_PASS):
            _zero_slice(zbuf, zr, acc_sh, r0, rpt)
            plsc.subcore_barrier()
            sp = _scalar(p)
            ep = _scalar(p + 1)
            s8 = (sp // 8) * 8
            total = ep - s8
            mine = wid * B
            nbt = jnp.maximum(0, (total - mine + NW * B - 1) // (NW * B))

            def body(m, _):
                kbase = pl.multiple_of(s8 + mine + m * (NW * B), 8)
                pltpu.sync_copy(v_hbm.at[pl.ds(kbase, B)], idxv_v)
                pltpu.sync_copy(he_hbm.at[pl.ds(kbase, B)], idxe_v)
                for t in range(B // 16):
                    e16 = idxe_v[pl.ds(t * 16, 16)]
                    loc = e16 - p * W_SEG
                    valid = (loc >= 0) & (loc < W_SEG)
                    idxl_v[pl.ds(t * 16, 16)] = jnp.where(valid, loc, W_SEG)
                pltpu.async_copy(table_hbm.at[idxv_v], rows_v, sem).wait()
                pltpu.sync_copy(rows_v, acc_sh.at[idxl_v], add=True)
                return _

            lax.fori_loop(0, nbt, body, 0)
            plsc.subcore_barrier()
            pltpu.sync_copy(acc_sh.at[pl.ds(r0, rpt)],
                            out_hbm.at[c, pl.ds(p * W_SEG + r0, rpt)])
            plsc.subcore_barrier()

    return k(table, v_pad, he_pad, bounds)


_R = 1000


def _full(shape):
    return pl.BlockSpec(shape, lambda i: (0,) * len(shape))


def _mlp_ln_tail(x, W2, b2, W3, b3, g, bln):
    x = jnp.maximum(x, 0.0)
    x = jnp.maximum(jnp.dot(x, W2, preferred_element_type=jnp.float32) + b2, 0.0)
    x = jnp.dot(x, W3, preferred_element_type=jnp.float32) + b3
    mu = jnp.mean(x, axis=1, keepdims=True)
    var = jnp.mean((x - mu) ** 2, axis=1, keepdims=True)
    return (x - mu) * lax.rsqrt(var + 1e-5) * g + bln


def _edge_mlp_g(xe2, he_feat, p):
    def body(xe_ref, he_ref, W1_ref, b1_ref, W2_ref, b2_ref, W3_ref, b3_ref,
             g_ref, bln_ref, newhe_ref, upde_ref):
        hef = he_ref[...]
        x = jnp.dot(xe_ref[...], W1_ref[0:256], preferred_element_type=jnp.float32)
        x = x + jnp.dot(hef, W1_ref[256:384], preferred_element_type=jnp.float32)
        x = x + b1_ref[...]
        u = _mlp_ln_tail(x, W2_ref[...], b2_ref[...], W3_ref[...], b3_ref[...],
                         g_ref[...], bln_ref[...])
        upde_ref[...] = u
        newhe_ref[...] = u + hef

    grid = N_EDGES // _R
    return pl.pallas_call(
        body,
        grid=(grid,),
        in_specs=[
            pl.BlockSpec((_R, 2 * D), lambda i: (i, 0)),
            pl.BlockSpec((_R, D), lambda i: (i, 0)),
            _full((3 * D, D)), _full((1, D)), _full((D, D)), _full((1, D)),
            _full((D, D)), _full((1, D)), _full((1, D)), _full((1, D)),
        ],
        out_specs=[pl.BlockSpec((_R, D), lambda i: (i, 0))] * 2,
        out_shape=[jax.ShapeDtypeStruct((N_EDGES, D), jnp.float32)] * 2,
    )(xe2, he_feat, p['edge_Ws'][0], p['edge_bs'][0].reshape(1, D),
      p['edge_Ws'][1], p['edge_bs'][1].reshape(1, D),
      p['edge_Ws'][2], p['edge_bs'][2].reshape(1, D),
      p['edge_ln_g'].reshape(1, D), p['edge_ln_b'].reshape(1, D))


def _edge_mlp_h(agg0, agg1, he_feat, p):
    def body(a0_ref, a1_ref, he_ref, W1_ref, b1_ref, W2_ref, b2_ref, W3_ref,
             b3_ref, g_ref, bln_ref, newhe_ref, upde_ref):
        hef = he_ref[...]
        agg = a0_ref[...] + a1_ref[...]
        x = jnp.dot(agg, W1_ref[0:128], preferred_element_type=jnp.float32)
        x = x + jnp.dot(hef, W1_ref[128:256], preferred_element_type=jnp.float32)
        x = x + b1_ref[...]
        u = _mlp_ln_tail(x, W2_ref[...], b2_ref[...], W3_ref[...], b3_ref[...],
                         g_ref[...], bln_ref[...])
        upde_ref[...] = u
        newhe_ref[...] = u + hef

    grid = N_EDGES // _R
    return pl.pallas_call(
        body,
        grid=(grid,),
        in_specs=[
            pl.BlockSpec((_R, D), lambda i: (i, 0)),
            pl.BlockSpec((_R, D), lambda i: (i, 0)),
            pl.BlockSpec((_R, D), lambda i: (i, 0)),
            _full((2 * D, D)), _full((1, D)), _full((D, D)), _full((1, D)),
            _full((D, D)), _full((1, D)), _full((1, D)), _full((1, D)),
        ],
        out_specs=[pl.BlockSpec((_R, D), lambda i: (i, 0))] * 2,
        out_shape=[jax.ShapeDtypeStruct((N_EDGES, D), jnp.float32)] * 2,
    )(agg0, agg1, he_feat, p['edge_Ws'][0], p['edge_bs'][0].reshape(1, D),
      p['edge_Ws'][1], p['edge_bs'][1].reshape(1, D),
      p['edge_Ws'][2], p['edge_bs'][2].reshape(1, D),
      p['edge_ln_g'].reshape(1, D), p['edge_ln_b'].reshape(1, D))


def _node_mlp(acc0, acc1, X, p, extra=None):
    has_extra = extra is not None

    def body(*refs):
        if has_extra:
            (a0_ref, a1_ref, x_ref, ex_ref, W1_ref, b1_ref, W2_ref, b2_ref,
             W3_ref, b3_ref, g_ref, bln_ref, out_ref) = refs
        else:
            (a0_ref, a1_ref, x_ref, W1_ref, b1_ref, W2_ref, b2_ref,
             W3_ref, b3_ref, g_ref, bln_ref, out_ref) = refs
        xv = a0_ref[...] + a1_ref[...]
        xx = x_ref[...]
        x = jnp.dot(xv, W1_ref[0:128], preferred_element_type=jnp.float32)
        x = x + jnp.dot(xx, W1_ref[128:256], preferred_element_type=jnp.float32)
        x = x + b1_ref[...]
        u = _mlp_ln_tail(x, W2_ref[...], b2_ref[...], W3_ref[...], b3_ref[...],
                         g_ref[...], bln_ref[...])
        out = u + xx
        if has_extra:
            out = out + ex_ref[...]
        out_ref[...] = out

    grid = N_NODES // _R
    row = pl.BlockSpec((_R, D), lambda i: (i, 0))
    in_specs = [row, row, row] + ([row] if has_extra else []) + [
        _full((2 * D, D)), _full((1, D)), _full((D, D)), _full((1, D)),
        _full((D, D)), _full((1, D)), _full((1, D)), _full((1, D)),
    ]
    args = [acc0, acc1, X] + ([extra] if has_extra else []) + [
        p['node_Ws'][0], p['node_bs'][0].reshape(1, D),
        p['node_Ws'][1], p['node_bs'][1].reshape(1, D),
        p['node_Ws'][2], p['node_bs'][2].reshape(1, D),
        p['node_ln_g'].reshape(1, D), p['node_ln_b'].reshape(1, D),
    ]
    return pl.pallas_call(
        body,
        grid=(grid,),
        in_specs=in_specs,
        out_specs=row,
        out_shape=jax.ShapeDtypeStruct((N_NODES, D), jnp.float32),
    )(*args)


def kernel(x_g, vertices_g, hyperedges_g, edge_features_g,
           x_h, vertices_h, hyperedges_h, edge_features_h, params):
    he_g = edge_features_g
    xg = x_g
    for p in params['g']:
        pairs = _sc_gather(xg, vertices_g)
        xe2 = pairs.reshape(N_EDGES, 2 * D)
        new_he, upd_e = _edge_mlp_g(xe2, he_g, p)
        acc = _sc_gather_scatter(upd_e, hyperedges_g, vertices_g)
        xg = _node_mlp(acc[0, :N_NODES], acc[1, :N_NODES], xg, p)
        he_g = new_he

    pad_v = jnp.zeros((SEG_PAD,), jnp.int32)
    pad_e = jnp.full((SEG_PAD,), jnp.int32(1 << 28))
    v_pad = jnp.concatenate([vertices_h, pad_v])
    he_pad = jnp.concatenate([hyperedges_h, pad_e])
    bounds = jnp.searchsorted(
        hyperedges_h, jnp.arange(N_PASS + 1, dtype=jnp.int32) * W_SEG
    ).astype(jnp.int32)
    bounds = jnp.concatenate(
        [bounds, jnp.full((BND_LEN - (N_PASS + 1),), 2 * N_EDGES, jnp.int32)])

    he_h = edge_features_h
    xh = x_h
    for i, p in enumerate(params['h']):
        parts = _sc_segsum(xh, v_pad, he_pad, bounds)
        new_he, upd_e = _edge_mlp_h(parts[0], parts[1], he_h, p)
        acc = _sc_gather_scatter(upd_e, hyperedges_h, vertices_h)
        extra = xg if i == len(params['h']) - 1 else None
        xh = _node_mlp(acc[0, :N_NODES], acc[1, :N_NODES], xh, p, extra=extra)
        he_h = new_he

    return (xh, he_g)


# speedup vs baseline: 4.6239x; 1.1838x over previous
import functools

import jax
import jax.numpy as jnp
from jax import lax
from jax.experimental import pallas as pl
from jax.experimental.pallas import tpu as pltpu
from jax.experimental.pallas import tpu_sc as plsc

N_NODES = 10000
N_EDGES = 160000
D = 128
NC = 2
NS = 16
NW = NC * NS
B = 400


def _sc_mesh():
    return plsc.VectorSubcoreMesh(core_axis_name="c", subcore_axis_name="s")


def _sc_gather(table, idx):
    n = idx.shape[0]
    per = n // NW
    nb = per // B

    @functools.partial(
        pl.kernel, mesh=_sc_mesh(),
        out_type=jax.ShapeDtypeStruct((n, D), jnp.float32),
        scratch_types=[
            pltpu.VMEM((B,), jnp.int32),
            pltpu.VMEM((B,), jnp.int32),
            pltpu.VMEM((B, D), jnp.float32),
            pltpu.VMEM((B, D), jnp.float32),
            pltpu.SemaphoreType.DMA,
            pltpu.SemaphoreType.DMA,
        ],
    )
    def k(table_hbm, idx_hbm, out_hbm, i0, i1, r0, r1, s0, s1):
        wid = lax.axis_index("s") * NC + lax.axis_index("c")
        base0 = wid * per
        idx_v, rows_v, sems = (i0, i1), (r0, r1), (s0, s1)
        gh = [None, None]
        for j in range(nb):
            b = j & 1
            o = 1 - b
            base = base0 + j * B
            pltpu.sync_copy(idx_hbm.at[pl.ds(base, B)], idx_v[b])
            gh[b] = pltpu.async_copy(table_hbm.at[idx_v[b]], rows_v[b], sems[b])
            if j:
                gh[o].wait()
                pltpu.sync_copy(rows_v[o], out_hbm.at[pl.ds(base - B, B)])
        b = (nb - 1) & 1
        gh[b].wait()
        pltpu.sync_copy(rows_v[b], out_hbm.at[pl.ds(base0 + (nb - 1) * B, B)])

    return k(table, idx)


def _zero_zbuf(zbuf, rows):
    def zrow(r, _):
        for c8 in range(D // 16):
            zbuf[r, pl.ds(c8 * 16, 16)] = jnp.zeros((16,), jnp.float32)
        return _
    lax.fori_loop(0, rows, zrow, 0)


def _zero_slice(zbuf, zr, dst, r0, rows):
    for off in range(0, rows, zr):
        cnt = min(zr, rows - off)
        src = zbuf if cnt == zr else zbuf.at[pl.ds(0, cnt)]
        pltpu.sync_copy(src, dst.at[pl.ds(r0 + off, cnt)])


def _sc_gather_scatter(table, idx_src, idx_dst):
    n = idx_src.shape[0]
    bk = 80
    per = n // NW
    nb = per // bk
    acc_rows = 10240
    rpt = acc_rows // NS

    @functools.partial(
        pl.kernel, mesh=_sc_mesh(),
        out_type=jax.ShapeDtypeStruct((NC, acc_rows, D), jnp.float32),
        scratch_types=[
            pltpu.VMEM((bk,), jnp.int32),
            pltpu.VMEM((bk,), jnp.int32),
            pltpu.VMEM((bk,), jnp.int32),
            pltpu.VMEM((bk,), jnp.int32),
            pltpu.VMEM((bk, D), jnp.float32),
            pltpu.VMEM((bk, D), jnp.float32),
            pltpu.VMEM_SHARED((acc_rows, D), jnp.float32),
            pltpu.SemaphoreType.DMA,
            pltpu.SemaphoreType.DMA,
        ],
    )
    def k(table_hbm, src_hbm, dst_hbm, out_hbm, is0, is1, id0, id1, r0v, r1v,
          acc_sh, sm0, sm1):
        c = lax.axis_index("c")
        s = lax.axis_index("s")
        wid = s * NC + c
        idxs_v, idxd_v = (is0, is1), (id0, id1)
        rows_v, sems = (r0v, r1v), (sm0, sm1)
        _zero_zbuf(r0v, bk)
        r0 = s * rpt
        _zero_slice(r0v, bk, acc_sh, r0, rpt)
        plsc.subcore_barrier()
        base0 = wid * per
        gh = [None, None]
        for j in range(nb):
            b = j & 1
            o = 1 - b
            base = base0 + j * bk
            pltpu.sync_copy(src_hbm.at[pl.ds(base, bk)], idxs_v[b])
            pltpu.sync_copy(dst_hbm.at[pl.ds(base, bk)], idxd_v[b])
            gh[b] = pltpu.async_copy(table_hbm.at[idxs_v[b]], rows_v[b], sems[b])
            if j:
                gh[o].wait()
                pltpu.sync_copy(rows_v[o], acc_sh.at[idxd_v[o]], add=True)
        b = (nb - 1) & 1
        gh[b].wait()
        pltpu.sync_copy(rows_v[b], acc_sh.at[idxd_v[b]], add=True)
        plsc.subcore_barrier()
        pltpu.sync_copy(acc_sh.at[pl.ds(r0, rpt)], out_hbm.at[c, pl.ds(r0, rpt)])

    return k(table, idx_src, idx_dst)


def _sc_scatter_pairs(table, vs, vr):
    ne = table.shape[0]
    be = 40
    per = ne // NW
    nb = per // be
    acc_rows = 10240
    rpt = acc_rows // NS

    @functools.partial(
        pl.kernel, mesh=_sc_mesh(),
        out_type=jax.ShapeDtypeStruct((NC, acc_rows, D), jnp.float32),
        scratch_types=[
            pltpu.VMEM((be,), jnp.int32),
            pltpu.VMEM((be,), jnp.int32),
            pltpu.VMEM((be,), jnp.int32),
            pltpu.VMEM((be,), jnp.int32),
            pltpu.VMEM((be, D), jnp.float32),
            pltpu.VMEM((be, D), jnp.float32),
            pltpu.VMEM_SHARED((acc_rows, D), jnp.float32),
            pltpu.SemaphoreType.DMA,
            pltpu.SemaphoreType.DMA,
        ],
    )
    def k(table_hbm, vs_hbm, vr_hbm, out_hbm, ia0, ia1, ib0, ib1, r0v, r1v,
          acc_sh, sm0, sm1):
        c = lax.axis_index("c")
        s = lax.axis_index("s")
        wid = s * NC + c
        idxa_v, idxb_v = (ia0, ia1), (ib0, ib1)
        rows_v, sems = (r0v, r1v), (sm0, sm1)
        _zero_zbuf(r0v, be)
        r0 = s * rpt
        _zero_slice(r0v, be, acc_sh, r0, rpt)
        plsc.subcore_barrier()
        base0 = wid * per
        gh = [None, None]

        def scatter(o):
            pltpu.sync_copy(rows_v[o], acc_sh.at[idxa_v[o]], add=True)
            pltpu.sync_copy(rows_v[o], acc_sh.at[idxb_v[o]], add=True)

        for j in range(nb):
            b = j & 1
            o = 1 - b
            base = base0 + j * be
            pltpu.sync_copy(vs_hbm.at[pl.ds(base, be)], idxa_v[b])
            pltpu.sync_copy(vr_hbm.at[pl.ds(base, be)], idxb_v[b])
            gh[b] = pltpu.async_copy(table_hbm.at[pl.ds(base, be)], rows_v[b],
                                     sems[b])
            if j:
                gh[o].wait()
                scatter(o)
        b = (nb - 1) & 1
        gh[b].wait()
        scatter(b)
        plsc.subcore_barrier()
        pltpu.sync_copy(acc_sh.at[pl.ds(r0, rpt)], out_hbm.at[c, pl.ds(r0, rpt)])

    return k(table, vs, vr)


W_SEG = 6400
N_PASS = N_EDGES // W_SEG
B_SEG = 192
SEG_PAD = 16000
BND_LEN = 48


def _sc_segsum(table, v_pad, he_pad, bounds):
    zr = 80
    rpt = W_SEG // NS
    stride = NS * B_SEG

    stride = NS * B_SEG

    @functools.partial(
        pl.kernel, mesh=_sc_mesh(),
        out_type=jax.ShapeDtypeStruct((N_EDGES, D), jnp.float32),
        scratch_types=[
            pltpu.VMEM((BND_LEN,), jnp.int32),
            pltpu.VMEM((B_SEG,), jnp.int32),
            pltpu.VMEM((B_SEG,), jnp.int32),
            pltpu.VMEM((B_SEG,), jnp.int32),
            pltpu.VMEM((B_SEG,), jnp.int32),
            pltpu.VMEM((B_SEG,), jnp.int32),
            pltpu.VMEM((B_SEG,), jnp.int32),
            pltpu.VMEM((B_SEG, D), jnp.float32),
            pltpu.VMEM((B_SEG, D), jnp.float32),
            pltpu.VMEM((zr, D), jnp.float32),
            pltpu.VMEM_SHARED((W_SEG + 8, D), jnp.float32),
            pltpu.SemaphoreType.DMA,
            pltpu.SemaphoreType.DMA,
        ],
    )
    def k(table_hbm, v_hbm, he_hbm, bounds_hbm, out_hbm, bnd_v, iva, ivb,
          iea, ieb, ila, ilb, ra, rb, zbuf, acc_sh, sma, smb):
        c = lax.axis_index("c")
        s = lax.axis_index("s")
        bufA = (iva, iea, ila, ra, sma)
        bufB = (ivb, ieb, ilb, rb, smb)
        _zero_zbuf(zbuf, zr)
        pltpu.sync_copy(bounds_hbm, bnd_v)

        def _scalar(i):
            return bnd_v[pl.ds(i, 16)][0]

        pstar = _scalar(N_PASS + 1)
        r0 = s * rpt
        mine = s * B_SEG
        for p in range(N_PASS):
            active = jnp.where(c == 0, p < pstar, p >= pstar)
            sp = _scalar(p)
            ep = _scalar(p + 1)
            s8 = (sp // 8) * 8
            total = ep - s8
            nbt = jnp.maximum(0, (total - mine + stride - 1) // stride)

            def load(j, buf, p=p, s8=s8):
                idxv, idxe, idxl, rows, sem = buf
                kbase = pl.multiple_of(s8 + mine + j * stride, 8)
                pltpu.sync_copy(v_hbm.at[pl.ds(kbase, B_SEG)], idxv)
                pltpu.sync_copy(he_hbm.at[pl.ds(kbase, B_SEG)], idxe)
                for t in range(B_SEG // 16):
                    e16 = idxe[pl.ds(t * 16, 16)]
                    loc = e16 - p * W_SEG
                    valid = (loc >= 0) & (loc < W_SEG)
                    idxl[pl.ds(t * 16, 16)] = jnp.where(valid, loc, W_SEG)
                return pltpu.async_copy(table_hbm.at[idxv], rows, sem)

            def scat(buf):
                _, _, idxl, rows, _ = buf
                pltpu.sync_copy(rows, acc_sh.at[idxl], add=True)

            @pl.when(active)
            def _zero():
                _zero_slice(zbuf, zr, acc_sh, r0, rpt)

            plsc.subcore_barrier()

            def body(m, _):
                ha = load(2 * m, bufA)
                hb = load(2 * m + 1, bufB)
                ha.wait()
                scat(bufA)
                hb.wait()
                scat(bufB)
                return _

            nbt_eff = jnp.where(active, nbt, 0)
            lax.fori_loop(0, (nbt_eff + 1) // 2, body, 0)

            plsc.subcore_barrier()

            @pl.when(active)
            def _dump():
                pltpu.sync_copy(acc_sh.at[pl.ds(r0, rpt)],
                                out_hbm.at[pl.ds(p * W_SEG + r0, rpt)])

            plsc.subcore_barrier()

    return k(table, v_pad, he_pad, bounds)


_R = 1000


def _full(shape):
    return pl.BlockSpec(shape, lambda i: (0,) * len(shape))


def _mlp_ln_tail(x, W2, b2, W3, b3, g, bln):
    x = jnp.maximum(x, 0.0)
    x = jnp.maximum(jnp.dot(x, W2, preferred_element_type=jnp.float32) + b2, 0.0)
    x = jnp.dot(x, W3, preferred_element_type=jnp.float32) + b3
    mu = jnp.mean(x, axis=1, keepdims=True)
    var = jnp.mean((x - mu) ** 2, axis=1, keepdims=True)
    return (x - mu) * lax.rsqrt(var + 1e-5) * g + bln


def _edge_mlp_g(xe2, he_feat, p):
    def body(xe_ref, he_ref, W1_ref, b1_ref, W2_ref, b2_ref, W3_ref, b3_ref,
             g_ref, bln_ref, newhe_ref, upde_ref):
        hef = he_ref[...]
        x = jnp.dot(xe_ref[...], W1_ref[0:256], preferred_element_type=jnp.float32)
        x = x + jnp.dot(hef, W1_ref[256:384], preferred_element_type=jnp.float32)
        x = x + b1_ref[...]
        u = _mlp_ln_tail(x, W2_ref[...], b2_ref[...], W3_ref[...], b3_ref[...],
                         g_ref[...], bln_ref[...])
        upde_ref[...] = u
        newhe_ref[...] = u + hef

    grid = N_EDGES // _R
    return pl.pallas_call(
        body,
        grid=(grid,),
        in_specs=[
            pl.BlockSpec((_R, 2 * D), lambda i: (i, 0)),
            pl.BlockSpec((_R, D), lambda i: (i, 0)),
            _full((3 * D, D)), _full((1, D)), _full((D, D)), _full((1, D)),
            _full((D, D)), _full((1, D)), _full((1, D)), _full((1, D)),
        ],
        out_specs=[pl.BlockSpec((_R, D), lambda i: (i, 0))] * 2,
        out_shape=[jax.ShapeDtypeStruct((N_EDGES, D), jnp.float32)] * 2,
    )(xe2, he_feat, p['edge_Ws'][0], p['edge_bs'][0].reshape(1, D),
      p['edge_Ws'][1], p['edge_bs'][1].reshape(1, D),
      p['edge_Ws'][2], p['edge_bs'][2].reshape(1, D),
      p['edge_ln_g'].reshape(1, D), p['edge_ln_b'].reshape(1, D))


def _edge_mlp_h(agg, he_feat, p):
    def body(a_ref, he_ref, W1_ref, b1_ref, W2_ref, b2_ref, W3_ref,
             b3_ref, g_ref, bln_ref, newhe_ref, upde_ref):
        hef = he_ref[...]
        x = jnp.dot(a_ref[...], W1_ref[0:128], preferred_element_type=jnp.float32)
        x = x + jnp.dot(hef, W1_ref[128:256], preferred_element_type=jnp.float32)
        x = x + b1_ref[...]
        u = _mlp_ln_tail(x, W2_ref[...], b2_ref[...], W3_ref[...], b3_ref[...],
                         g_ref[...], bln_ref[...])
        upde_ref[...] = u
        newhe_ref[...] = u + hef

    grid = N_EDGES // _R
    return pl.pallas_call(
        body,
        grid=(grid,),
        in_specs=[
            pl.BlockSpec((_R, D), lambda i: (i, 0)),
            pl.BlockSpec((_R, D), lambda i: (i, 0)),
            _full((2 * D, D)), _full((1, D)), _full((D, D)), _full((1, D)),
            _full((D, D)), _full((1, D)), _full((1, D)), _full((1, D)),
        ],
        out_specs=[pl.BlockSpec((_R, D), lambda i: (i, 0))] * 2,
        out_shape=[jax.ShapeDtypeStruct((N_EDGES, D), jnp.float32)] * 2,
    )(agg, he_feat, p['edge_Ws'][0], p['edge_bs'][0].reshape(1, D),
      p['edge_Ws'][1], p['edge_bs'][1].reshape(1, D),
      p['edge_Ws'][2], p['edge_bs'][2].reshape(1, D),
      p['edge_ln_g'].reshape(1, D), p['edge_ln_b'].reshape(1, D))


def _node_mlp(acc0, acc1, X, p, extra=None):
    has_extra = extra is not None

    def body(*refs):
        if has_extra:
            (a0_ref, a1_ref, x_ref, ex_ref, W1_ref, b1_ref, W2_ref, b2_ref,
             W3_ref, b3_ref, g_ref, bln_ref, out_ref) = refs
        else:
            (a0_ref, a1_ref, x_ref, W1_ref, b1_ref, W2_ref, b2_ref,
             W3_ref, b3_ref, g_ref, bln_ref, out_ref) = refs
        xv = a0_ref[...] + a1_ref[...]
        xx = x_ref[...]
        x = jnp.dot(xv, W1_ref[0:128], preferred_element_type=jnp.float32)
        x = x + jnp.dot(xx, W1_ref[128:256], preferred_element_type=jnp.float32)
        x = x + b1_ref[...]
        u = _mlp_ln_tail(x, W2_ref[...], b2_ref[...], W3_ref[...], b3_ref[...],
                         g_ref[...], bln_ref[...])
        out = u + xx
        if has_extra:
            out = out + ex_ref[...]
        out_ref[...] = out

    grid = N_NODES // _R
    row = pl.BlockSpec((_R, D), lambda i: (i, 0))
    in_specs = [row, row, row] + ([row] if has_extra else []) + [
        _full((2 * D, D)), _full((1, D)), _full((D, D)), _full((1, D)),
        _full((D, D)), _full((1, D)), _full((1, D)), _full((1, D)),
    ]
    args = [acc0, acc1, X] + ([extra] if has_extra else []) + [
        p['node_Ws'][0], p['node_bs'][0].reshape(1, D),
        p['node_Ws'][1], p['node_bs'][1].reshape(1, D),
        p['node_Ws'][2], p['node_bs'][2].reshape(1, D),
        p['node_ln_g'].reshape(1, D), p['node_ln_b'].reshape(1, D),
    ]
    return pl.pallas_call(
        body,
        grid=(grid,),
        in_specs=in_specs,
        out_specs=row,
        out_shape=jax.ShapeDtypeStruct((N_NODES, D), jnp.float32),
    )(*args)


def kernel(x_g, vertices_g, hyperedges_g, edge_features_g,
           x_h, vertices_h, hyperedges_h, edge_features_h, params):
    he_g = edge_features_g
    xg = x_g
    vs = vertices_g[0::2].astype(jnp.int32)
    vr = vertices_g[1::2].astype(jnp.int32)
    for p in params['g']:
        pairs = _sc_gather(xg, vertices_g)
        xe2 = pairs.reshape(N_EDGES, 2 * D)
        new_he, upd_e = _edge_mlp_g(xe2, he_g, p)
        acc = _sc_scatter_pairs(upd_e, vs, vr)
        xg = _node_mlp(acc[0, :N_NODES], acc[1, :N_NODES], xg, p)
        he_g = new_he

    pad_v = jnp.zeros((SEG_PAD,), jnp.int32)
    pad_e = jnp.full((SEG_PAD,), jnp.int32(1 << 28))
    v_pad = jnp.concatenate([vertices_h, pad_v])
    he_pad = jnp.concatenate([hyperedges_h, pad_e])
    bounds = jnp.searchsorted(
        hyperedges_h, jnp.arange(N_PASS + 1, dtype=jnp.int32) * W_SEG
    ).astype(jnp.int32)
    pstar = jnp.argmin(jnp.abs(bounds - N_EDGES)).astype(jnp.int32)
    bounds = jnp.concatenate(
        [bounds, pstar[None],
         jnp.full((BND_LEN - (N_PASS + 2),), 2 * N_EDGES, jnp.int32)])

    he_h = edge_features_h
    xh = x_h
    for i, p in enumerate(params['h']):
        agg = _sc_segsum(xh, v_pad, he_pad, bounds)
        new_he, upd_e = _edge_mlp_h(agg, he_h, p)
        acc = _sc_gather_scatter(upd_e, hyperedges_h, vertices_h)
        extra = xg if i == len(params['h']) - 1 else None
        xh = _node_mlp(acc[0, :N_NODES], acc[1, :N_NODES], xh, p, extra=extra)
        he_h = new_he

    return (xh, he_g)
